# Initial kernel scaffold; baseline (speedup 1.0000x reference)
import jax
import jax.numpy as jnp
from jax.experimental import pallas as pl


def kernel(token_ids, weight):
    raise NotImplementedError("write your pallas kernel here")


# speedup vs baseline: 1.8402x; 1.8402x over previous
import functools

import jax
import jax.numpy as jnp
from jax import lax
from jax.experimental import pallas as pl
from jax.experimental.pallas import tpu as pltpu
from jax.experimental.pallas import tpu_sc as plsc

D = 64
CHUNK = 128
NBUF = 2


@functools.lru_cache(maxsize=None)
def _make(nw, nch):
    mesh = plsc.VectorSubcoreMesh(core_axis_name="c", subcore_axis_name="s")
    nc = plsc.get_sparse_core_info().num_cores

    @functools.partial(
        pl.kernel,
        mesh=mesh,
        compiler_params=pltpu.CompilerParams(use_tc_tiling_on_sc=False),
        out_type=jax.ShapeDtypeStruct((nw, nch, CHUNK, D), jnp.float32),
        scratch_types=[
            pltpu.VMEM((nch, CHUNK), jnp.int32),
            pltpu.VMEM((NBUF, CHUNK, D), jnp.float32),
            pltpu.SemaphoreType.DMA,
            pltpu.SemaphoreType.DMA,
        ],
    )
    def k(idx_hbm, table_hbm, out_hbm, idx_v, rows_v, gsem, wsem):
        wid = lax.axis_index("s") * nc + lax.axis_index("c")
        pltpu.sync_copy(idx_hbm.at[wid], idx_v)

        pltpu.async_copy(table_hbm.at[idx_v.at[0]], rows_v.at[0], gsem)

        def body(i, _):
            for b in range(NBUF):
                j = i * NBUF + b
                nxt = j + 1
                @pl.when(nxt < nch)
                def _():
                    pltpu.make_async_copy(
                        table_hbm.at[idx_v.at[nxt]], rows_v.at[(b + 1) % NBUF], gsem
                    ).start()
                pltpu.make_async_copy(
                    table_hbm.at[idx_v.at[j]], rows_v.at[b], gsem
                ).wait()
                pltpu.sync_copy(rows_v.at[b], out_hbm.at[wid, j])
            return 0

        lax.fori_loop(0, nch // NBUF, body, 0)

    return k


def kernel(token_ids, weight):
    batch, hist = token_ids.shape
    total = batch * hist
    nw = 32
    assert total % (nw * CHUNK) == 0
    nch = total // (nw * CHUNK)
    idx = token_ids.reshape(nw, nch, CHUNK).astype(jnp.int32)
    out = _make(nw, nch)(idx, weight)
    return out.reshape(batch, hist, D)


# speedup vs baseline: 1.8749x; 1.0189x over previous
import functools

import jax
import jax.numpy as jnp
from jax import lax
from jax.experimental import pallas as pl
from jax.experimental.pallas import tpu as pltpu
from jax.experimental.pallas import tpu_sc as plsc

D = 64
CHUNK = 128
K = 5


@functools.lru_cache(maxsize=None)
def _make(nw, ngrp):
    mesh = plsc.VectorSubcoreMesh(core_axis_name="c", subcore_axis_name="s")
    nc = plsc.get_sparse_core_info().num_cores
    nch = ngrp * K

    @functools.partial(
        pl.kernel,
        mesh=mesh,
        compiler_params=pltpu.CompilerParams(use_tc_tiling_on_sc=False),
        out_type=jax.ShapeDtypeStruct((nw, ngrp, K, CHUNK, D), jnp.float32),
        scratch_types=[
            pltpu.VMEM((nch, CHUNK), jnp.int32),
            pltpu.VMEM((2, K, CHUNK, D), jnp.float32),
            pltpu.SemaphoreType.DMA,
        ],
    )
    def k(idx_hbm, table_hbm, out_hbm, idx_v, rows_v, gsem):
        wid = lax.axis_index("s") * nc + lax.axis_index("c")
        pltpu.sync_copy(idx_hbm.at[wid], idx_v)

        def fire(g, b):
            for kk in range(K):
                pltpu.make_async_copy(
                    table_hbm.at[idx_v.at[g * K + kk]], rows_v.at[b, kk], gsem
                ).start()

        def drain(b):
            for kk in range(K):
                pltpu.make_async_copy(
                    table_hbm.at[idx_v.at[0]], rows_v.at[b, kk], gsem
                ).wait()

        fire(0, 0)

        def body(i, _):
            for b in range(2):
                g = i * 2 + b

                @pl.when(g + 1 < ngrp)
                def _():
                    fire(g + 1, 1 - b)

                drain(b)
                pltpu.sync_copy(rows_v.at[b], out_hbm.at[wid, g])
            return 0

        lax.fori_loop(0, ngrp // 2, body, 0)

    return k


def kernel(token_ids, weight):
    batch, hist = token_ids.shape
    total = batch * hist
    nw = 32
    grp = nw * CHUNK * K
    assert total % grp == 0 and (total // grp) % 2 == 0
    ngrp = total // grp
    idx = token_ids.reshape(nw, ngrp * K, CHUNK).astype(jnp.int32)
    out = _make(nw, ngrp)(idx, weight)
    return out.reshape(batch, hist, D)
